# baseline (speedup 1.0000x reference)
import jax
import jax.numpy as jnp
from jax.experimental import pallas as pl


def _fkin_body(d_ref, out_ref):
    d = d_ref[...]
    out_ref[...] = d[:, 0:3] * jnp.cos(d[:, 3:6]) + jnp.sin(d[:, 6:9])


def kernel(masked_dofs, full_dofs, pose_coords, w, mask_idx, kin_id):
    n = masked_dofs.shape[0]
    nflat = full_dofs.size
    winner = jnp.full((nflat,), -1, jnp.int32).at[mask_idx].max(
        jnp.arange(n, dtype=jnp.int32))
    dofs_flat = jnp.where(winner >= 0,
                          masked_dofs[jnp.clip(winner, 0)],
                          full_dofs.reshape(-1))
    dofs = dofs_flat.reshape(-1, 9)
    knodes = dofs.shape[0]
    blk = 8000
    kc = pl.pallas_call(
        _fkin_body,
        grid=(knodes // blk,),
        in_specs=[pl.BlockSpec((blk, 9), lambda i: (i, 0))],
        out_specs=pl.BlockSpec((blk, 3), lambda i: (i, 0)),
        out_shape=jax.ShapeDtypeStruct((knodes, 3), jnp.float32),
    )(dofs)
    k = kin_id.shape[0]
    ncoord = pose_coords.shape[0] * pose_coords.shape[1]
    winner2 = jnp.full((ncoord,), -1, jnp.int32).at[kin_id].max(
        jnp.arange(k, dtype=jnp.int32))
    coords_flat = jnp.where((winner2 >= 0)[:, None],
                            kc[jnp.clip(winner2, 0)],
                            pose_coords.reshape(-1, 3))
    coords = coords_flat.reshape(pose_coords.shape)
    return jnp.einsum("pad,d->p", coords * coords, w)


# speedup vs baseline: 22.5554x; 22.5554x over previous
import functools

import jax
import jax.numpy as jnp
from jax import lax
from jax.experimental import pallas as pl
from jax.experimental.pallas import tpu as pltpu, tpu_sc as plsc

_NDOF = 7_200_000
_NUPD = 2_400_000
_WIN = 112_640
_NWIN = 64
_LAST_WIN = _NDOF - 63 * _WIN
_UC = 3_840
_NCHUNK = _NUPD // _UC

_NROW = 800_000
_RPW = _NROW // 32
_WPW = _RPW * 3
_UC2 = 3_200
_NCHUNK2 = _NROW // _UC2

_mesh = plsc.VectorSubcoreMesh(core_axis_name="c", subcore_axis_name="s")
_sc_params = pltpu.CompilerParams(needs_layout_passes=False)


def _scan_updates(idx_hbm, val_hbm, idxv, valv, win, base):
    def chunk_body(c, carry):
        coff = pl.multiple_of(c * _UC, _UC)
        pltpu.sync_copy(idx_hbm.at[pl.ds(coff, _UC)], idxv)
        pltpu.sync_copy(val_hbm.at[pl.ds(coff, _UC)], valv)

        def vreg_body(u, carry2):
            off = pl.multiple_of(u * 16, 16)
            iv = idxv[pl.ds(off, 16)]
            local = iv - base
            um = plsc.bitcast(local, jnp.uint32) < jnp.uint32(_WIN)
            li = jnp.where(um, local, 0)
            vv = valv[pl.ds(off, 16)]
            plsc.store_scatter(win, [li], vv, mask=um)
            return carry2

        return lax.fori_loop(0, _UC // 16, vreg_body, carry)

    lax.fori_loop(0, _NCHUNK, chunk_body, 0)


@functools.partial(
    pl.kernel, mesh=_mesh,
    out_type=jax.ShapeDtypeStruct((_NDOF,), jnp.float32),
    scratch_types=[
        pltpu.VMEM((_WIN,), jnp.float32),
        pltpu.VMEM((_UC,), jnp.int32),
        pltpu.VMEM((_UC,), jnp.float32),
    ],
    compiler_params=_sc_params,
)
def _sc_dof_scatter(dofs_in, idx_hbm, val_hbm, dofs_out, win, idxv, valv):
    wid = lax.axis_index("s") * 2 + lax.axis_index("c")

    base0 = wid * _WIN
    pltpu.sync_copy(dofs_in.at[pl.ds(base0, _WIN)], win)
    _scan_updates(idx_hbm, val_hbm, idxv, valv, win, base0)
    pltpu.sync_copy(win, dofs_out.at[pl.ds(base0, _WIN)])

    base1 = (32 + wid) * _WIN

    @pl.when(wid != 31)
    def _():
        pltpu.sync_copy(dofs_in.at[pl.ds(base1, _WIN)], win)
        _scan_updates(idx_hbm, val_hbm, idxv, valv, win, base1)
        pltpu.sync_copy(win, dofs_out.at[pl.ds(base1, _WIN)])

    @pl.when(wid == 31)
    def _():
        pltpu.sync_copy(dofs_in.at[pl.ds(63 * _WIN, _LAST_WIN)],
                        win.at[pl.ds(0, _LAST_WIN)])
        _scan_updates(idx_hbm, val_hbm, idxv, valv, win, base1)
        pltpu.sync_copy(win.at[pl.ds(0, _LAST_WIN)],
                        dofs_out.at[pl.ds(63 * _WIN, _LAST_WIN)])


@functools.partial(
    pl.kernel, mesh=_mesh,
    out_type=jax.ShapeDtypeStruct((_NROW * 3,), jnp.float32),
    scratch_types=[
        pltpu.VMEM((_WPW,), jnp.float32),
        pltpu.VMEM((_UC2,), jnp.int32),
        pltpu.VMEM((_UC2,), jnp.float32),
        pltpu.VMEM((_UC2,), jnp.float32),
        pltpu.VMEM((_UC2,), jnp.float32),
    ],
    compiler_params=_sc_params,
)
def _sc_coord_scatter(coords_in, kin_hbm, kx_hbm, ky_hbm, kz_hbm, coords_out,
                      win, kinv, kxv, kyv, kzv):
    wid = lax.axis_index("s") * 2 + lax.axis_index("c")
    base_row = wid * _RPW
    wbase = wid * _WPW
    pltpu.sync_copy(coords_in.at[pl.ds(wbase, _WPW)], win)

    def chunk_body(c, carry):
        coff = pl.multiple_of(c * _UC2, _UC2)
        pltpu.sync_copy(kin_hbm.at[pl.ds(coff, _UC2)], kinv)
        pltpu.sync_copy(kx_hbm.at[pl.ds(coff, _UC2)], kxv)
        pltpu.sync_copy(ky_hbm.at[pl.ds(coff, _UC2)], kyv)
        pltpu.sync_copy(kz_hbm.at[pl.ds(coff, _UC2)], kzv)

        def vreg_body(u, carry2):
            off = pl.multiple_of(u * 16, 16)
            kid = kinv[pl.ds(off, 16)]
            lr = kid - base_row
            um = plsc.bitcast(lr, jnp.uint32) < jnp.uint32(_RPW)
            b3 = jnp.where(um, lr * 3, 0)
            plsc.store_scatter(win, [b3], kxv[pl.ds(off, 16)], mask=um)
            plsc.store_scatter(win, [b3 + 1], kyv[pl.ds(off, 16)], mask=um)
            plsc.store_scatter(win, [b3 + 2], kzv[pl.ds(off, 16)], mask=um)
            return carry2

        return lax.fori_loop(0, _UC2 // 16, vreg_body, carry)

    lax.fori_loop(0, _NCHUNK2, chunk_body, 0)
    pltpu.sync_copy(win, coords_out.at[pl.ds(wbase, _WPW)])


def _fkin_body(d_ref, out_ref):
    d = d_ref[...]
    out_ref[...] = d[:, 0:3] * jnp.cos(d[:, 3:6]) + jnp.sin(d[:, 6:9])


def kernel(masked_dofs, full_dofs, pose_coords, w, mask_idx, kin_id):
    dofs_flat = _sc_dof_scatter(full_dofs.reshape(-1), mask_idx, masked_dofs)
    dofs = dofs_flat.reshape(-1, 9)
    knodes = dofs.shape[0]
    blk = 8000
    kc = pl.pallas_call(
        _fkin_body,
        grid=(knodes // blk,),
        in_specs=[pl.BlockSpec((blk, 9), lambda i: (i, 0))],
        out_specs=pl.BlockSpec((blk, 3), lambda i: (i, 0)),
        out_shape=jax.ShapeDtypeStruct((knodes, 3), jnp.float32),
    )(dofs)
    coords_flat = _sc_coord_scatter(
        pose_coords.reshape(-1), kin_id,
        kc[:, 0], kc[:, 1], kc[:, 2])
    coords = coords_flat.reshape(pose_coords.shape)
    coords = lax.optimization_barrier(coords)
    return jnp.einsum("pad,d->p", coords * coords, w)


# speedup vs baseline: 29.9557x; 1.3281x over previous
import functools

import jax
import jax.numpy as jnp
from jax import lax
from jax.experimental import pallas as pl
from jax.experimental.pallas import tpu as pltpu, tpu_sc as plsc

_NDOF = 7_200_000
_NUPD = 2_400_000
_WIN = 112_640
_NWIN = 64
_LAST_WIN = _NDOF - 63 * _WIN
_UC = 3_200
_NCHUNK = _NUPD // _UC

_NROW = 800_000
_RPW = _NROW // 32
_WPW = _RPW * 3
_UC2 = 3_200
_NCHUNK2 = _NROW // _UC2

_mesh = plsc.VectorSubcoreMesh(core_axis_name="c", subcore_axis_name="s")
_sc_params = pltpu.CompilerParams(needs_layout_passes=False)


def _proc_chunk(idxr, valr, win, base):
    def vreg_body(u, carry):
        off0 = pl.multiple_of(u * 64, 64)
        for s in range(4):
            off = off0 + 16 * s
            iv = idxr[pl.ds(off, 16)]
            local = iv - base
            um = plsc.bitcast(local, jnp.uint32) < jnp.uint32(_WIN)
            li = jnp.where(um, local, 0)
            vv = valr[pl.ds(off, 16)]
            plsc.store_scatter(win, [li], vv, mask=um)
        return carry

    lax.fori_loop(0, _UC // 64, vreg_body, 0)


def _scan_updates(idx_hbm, val_hbm, bufs, win, base):
    def _start(c, slot):
        idxr, valr, si, sv = bufs[slot]
        coff = pl.multiple_of(c * _UC, _UC)
        pltpu.async_copy(idx_hbm.at[pl.ds(coff, _UC)], idxr, si)
        pltpu.async_copy(val_hbm.at[pl.ds(coff, _UC)], valr, sv)

    def _wait(c, slot):
        idxr, valr, si, sv = bufs[slot]
        coff = pl.multiple_of(c * _UC, _UC)
        pltpu.make_async_copy(idx_hbm.at[pl.ds(coff, _UC)], idxr, si).wait()
        pltpu.make_async_copy(val_hbm.at[pl.ds(coff, _UC)], valr, sv).wait()

    npair = _NCHUNK // 2
    _start(0, 0)

    def pair_body(i, carry):
        c0 = i * 2
        _wait(c0, 0)
        _start(c0 + 1, 1)
        _proc_chunk(bufs[0][0], bufs[0][1], win, base)
        _wait(c0 + 1, 1)

        @pl.when(i < npair - 1)
        def _():
            _start(c0 + 2, 0)

        _proc_chunk(bufs[1][0], bufs[1][1], win, base)
        return carry

    lax.fori_loop(0, npair, pair_body, 0)


@functools.partial(
    pl.kernel, mesh=_mesh,
    out_type=jax.ShapeDtypeStruct((_NDOF,), jnp.float32),
    scratch_types=[
        pltpu.VMEM((_WIN,), jnp.float32),
        pltpu.VMEM((_UC,), jnp.int32),
        pltpu.VMEM((_UC,), jnp.float32),
        pltpu.VMEM((_UC,), jnp.int32),
        pltpu.VMEM((_UC,), jnp.float32),
        pltpu.SemaphoreType.DMA,
        pltpu.SemaphoreType.DMA,
        pltpu.SemaphoreType.DMA,
        pltpu.SemaphoreType.DMA,
    ],
    compiler_params=_sc_params,
)
def _sc_dof_scatter(dofs_in, idx_hbm, val_hbm, dofs_out, win, idxv0, valv0,
                    idxv1, valv1, si0, sv0, si1, sv1):
    bufs = ((idxv0, valv0, si0, sv0), (idxv1, valv1, si1, sv1))
    wid = lax.axis_index("s") * 2 + lax.axis_index("c")

    base0 = wid * _WIN
    pltpu.sync_copy(dofs_in.at[pl.ds(base0, _WIN)], win)
    _scan_updates(idx_hbm, val_hbm, bufs, win, base0)
    pltpu.sync_copy(win, dofs_out.at[pl.ds(base0, _WIN)])

    base1 = (32 + wid) * _WIN

    @pl.when(wid != 31)
    def _():
        pltpu.sync_copy(dofs_in.at[pl.ds(base1, _WIN)], win)
        _scan_updates(idx_hbm, val_hbm, bufs, win, base1)
        pltpu.sync_copy(win, dofs_out.at[pl.ds(base1, _WIN)])

    @pl.when(wid == 31)
    def _():
        pltpu.sync_copy(dofs_in.at[pl.ds(63 * _WIN, _LAST_WIN)],
                        win.at[pl.ds(0, _LAST_WIN)])
        _scan_updates(idx_hbm, val_hbm, bufs, win, base1)
        pltpu.sync_copy(win.at[pl.ds(0, _LAST_WIN)],
                        dofs_out.at[pl.ds(63 * _WIN, _LAST_WIN)])


@functools.partial(
    pl.kernel, mesh=_mesh,
    out_type=jax.ShapeDtypeStruct((_NROW * 3,), jnp.float32),
    scratch_types=[
        pltpu.VMEM((_WPW,), jnp.float32),
        pltpu.VMEM((_UC2,), jnp.int32),
        pltpu.VMEM((_UC2,), jnp.float32),
        pltpu.VMEM((_UC2,), jnp.float32),
        pltpu.VMEM((_UC2,), jnp.float32),
        pltpu.VMEM((_UC2,), jnp.int32),
        pltpu.VMEM((_UC2,), jnp.float32),
        pltpu.VMEM((_UC2,), jnp.float32),
        pltpu.VMEM((_UC2,), jnp.float32),
        pltpu.SemaphoreType.DMA,
        pltpu.SemaphoreType.DMA,
    ],
    compiler_params=_sc_params,
)
def _sc_coord_scatter(coords_in, kin_hbm, kx_hbm, ky_hbm, kz_hbm, coords_out,
                      win, kinv0, kxv0, kyv0, kzv0, kinv1, kxv1, kyv1, kzv1,
                      s0, s1):
    wid = lax.axis_index("s") * 2 + lax.axis_index("c")
    base_row = wid * _RPW
    wbase = wid * _WPW
    pltpu.sync_copy(coords_in.at[pl.ds(wbase, _WPW)], win)
    sems = (s0, s1)
    srcs = (kin_hbm, kx_hbm, ky_hbm, kz_hbm)
    slot_bufs = ((kinv0, kxv0, kyv0, kzv0), (kinv1, kxv1, kyv1, kzv1))

    def _bufs(slot):
        return slot_bufs[slot]

    def _start(c, slot):
        coff = pl.multiple_of(c * _UC2, _UC2)
        for src, dst in zip(srcs, _bufs(slot)):
            pltpu.async_copy(src.at[pl.ds(coff, _UC2)], dst, sems[slot])

    def _wait(c, slot):
        coff = pl.multiple_of(c * _UC2, _UC2)
        for src, dst in zip(srcs, _bufs(slot)):
            pltpu.make_async_copy(src.at[pl.ds(coff, _UC2)], dst,
                                  sems[slot]).wait()

    def _proc(slot):
        kinr, kxr, kyr, kzr = _bufs(slot)

        def vreg_body(u, carry):
            off0 = pl.multiple_of(u * 64, 64)
            for s in range(4):
                off = off0 + 16 * s
                kid = kinr[pl.ds(off, 16)]
                lr = kid - base_row
                um = plsc.bitcast(lr, jnp.uint32) < jnp.uint32(_RPW)
                b3 = jnp.where(um, lr * 3, 0)
                plsc.store_scatter(win, [b3], kxr[pl.ds(off, 16)], mask=um)
                plsc.store_scatter(win, [b3 + 1], kyr[pl.ds(off, 16)], mask=um)
                plsc.store_scatter(win, [b3 + 2], kzr[pl.ds(off, 16)], mask=um)
            return carry

        lax.fori_loop(0, _UC2 // 64, vreg_body, 0)

    npair = _NCHUNK2 // 2
    _start(0, 0)

    def pair_body(i, carry):
        c0 = i * 2
        _wait(c0, 0)
        _start(c0 + 1, 1)
        _proc(0)
        _wait(c0 + 1, 1)

        @pl.when(i < npair - 1)
        def _():
            _start(c0 + 2, 0)

        _proc(1)
        return carry

    lax.fori_loop(0, npair, pair_body, 0)
    pltpu.sync_copy(win, coords_out.at[pl.ds(wbase, _WPW)])


def _fkin_body(d_ref, out_ref):
    d = d_ref[...]
    out_ref[...] = d[:, 0:3] * jnp.cos(d[:, 3:6]) + jnp.sin(d[:, 6:9])


def kernel(masked_dofs, full_dofs, pose_coords, w, mask_idx, kin_id):
    dofs_flat = _sc_dof_scatter(full_dofs.reshape(-1), mask_idx, masked_dofs)
    dofs = dofs_flat.reshape(-1, 9)
    knodes = dofs.shape[0]
    blk = 8000
    kc = pl.pallas_call(
        _fkin_body,
        grid=(knodes // blk,),
        in_specs=[pl.BlockSpec((blk, 9), lambda i: (i, 0))],
        out_specs=pl.BlockSpec((blk, 3), lambda i: (i, 0)),
        out_shape=jax.ShapeDtypeStruct((knodes, 3), jnp.float32),
    )(dofs)
    coords_flat = _sc_coord_scatter(
        pose_coords.reshape(-1), kin_id,
        kc[:, 0], kc[:, 1], kc[:, 2])
    coords = coords_flat.reshape(pose_coords.shape)
    coords = lax.optimization_barrier(coords)
    return jnp.einsum("pad,d->p", coords * coords, w)


# speedup vs baseline: 30.2554x; 1.0100x over previous
import functools

import jax
import jax.numpy as jnp
from jax import lax
from jax.experimental import pallas as pl
from jax.experimental.pallas import tpu as pltpu, tpu_sc as plsc

_NDOF = 7_200_000
_NUPD = 2_400_000
_WIN = 112_640
_NWIN = 64
_LAST_WIN = _NDOF - 63 * _WIN
_UC = 3_200
_NCHUNK = _NUPD // _UC

_NROW = 800_000
_RPW = _NROW // 32
_WPW = _RPW * 3
_UC2 = 3_200
_NCHUNK2 = _NROW // _UC2

_mesh = plsc.VectorSubcoreMesh(core_axis_name="c", subcore_axis_name="s")
_sc_params = pltpu.CompilerParams(needs_layout_passes=False)


def _proc_chunk(idxr, valr, win, base):
    def vreg_body(u, carry):
        off0 = pl.multiple_of(u * 128, 128)
        for s in range(8):
            off = off0 + 16 * s
            iv = idxr[pl.ds(off, 16)]
            local = iv - base
            um = plsc.bitcast(local, jnp.uint32) < jnp.uint32(_WIN)
            vv = valr[pl.ds(off, 16)]
            plsc.store_scatter(win, [local], vv, mask=um)
        return carry

    lax.fori_loop(0, _UC // 128, vreg_body, 0)


def _scan_updates(idx_hbm, val_hbm, bufs, win, base):
    def _start(c, slot):
        idxr, valr, si, sv = bufs[slot]
        coff = pl.multiple_of(c * _UC, _UC)
        pltpu.async_copy(idx_hbm.at[pl.ds(coff, _UC)], idxr, si)
        pltpu.async_copy(val_hbm.at[pl.ds(coff, _UC)], valr, sv)

    def _wait(c, slot):
        idxr, valr, si, sv = bufs[slot]
        coff = pl.multiple_of(c * _UC, _UC)
        pltpu.make_async_copy(idx_hbm.at[pl.ds(coff, _UC)], idxr, si).wait()
        pltpu.make_async_copy(val_hbm.at[pl.ds(coff, _UC)], valr, sv).wait()

    npair = _NCHUNK // 2
    _start(0, 0)

    def pair_body(i, carry):
        c0 = i * 2
        _wait(c0, 0)
        _start(c0 + 1, 1)
        _proc_chunk(bufs[0][0], bufs[0][1], win, base)
        _wait(c0 + 1, 1)

        @pl.when(i < npair - 1)
        def _():
            _start(c0 + 2, 0)

        _proc_chunk(bufs[1][0], bufs[1][1], win, base)
        return carry

    lax.fori_loop(0, npair, pair_body, 0)


@functools.partial(
    pl.kernel, mesh=_mesh,
    out_type=jax.ShapeDtypeStruct((_NDOF,), jnp.float32),
    scratch_types=[
        pltpu.VMEM((_WIN,), jnp.float32),
        pltpu.VMEM((_UC,), jnp.int32),
        pltpu.VMEM((_UC,), jnp.float32),
        pltpu.VMEM((_UC,), jnp.int32),
        pltpu.VMEM((_UC,), jnp.float32),
        pltpu.SemaphoreType.DMA,
        pltpu.SemaphoreType.DMA,
        pltpu.SemaphoreType.DMA,
        pltpu.SemaphoreType.DMA,
    ],
    compiler_params=_sc_params,
)
def _sc_dof_scatter(dofs_in, idx_hbm, val_hbm, dofs_out, win, idxv0, valv0,
                    idxv1, valv1, si0, sv0, si1, sv1):
    bufs = ((idxv0, valv0, si0, sv0), (idxv1, valv1, si1, sv1))
    wid = lax.axis_index("s") * 2 + lax.axis_index("c")

    base0 = wid * _WIN
    pltpu.sync_copy(dofs_in.at[pl.ds(base0, _WIN)], win)
    _scan_updates(idx_hbm, val_hbm, bufs, win, base0)
    pltpu.sync_copy(win, dofs_out.at[pl.ds(base0, _WIN)])

    base1 = (32 + wid) * _WIN

    @pl.when(wid != 31)
    def _():
        pltpu.sync_copy(dofs_in.at[pl.ds(base1, _WIN)], win)
        _scan_updates(idx_hbm, val_hbm, bufs, win, base1)
        pltpu.sync_copy(win, dofs_out.at[pl.ds(base1, _WIN)])

    @pl.when(wid == 31)
    def _():
        pltpu.sync_copy(dofs_in.at[pl.ds(63 * _WIN, _LAST_WIN)],
                        win.at[pl.ds(0, _LAST_WIN)])
        _scan_updates(idx_hbm, val_hbm, bufs, win, base1)
        pltpu.sync_copy(win.at[pl.ds(0, _LAST_WIN)],
                        dofs_out.at[pl.ds(63 * _WIN, _LAST_WIN)])


@functools.partial(
    pl.kernel, mesh=_mesh,
    out_type=jax.ShapeDtypeStruct((_NROW * 3,), jnp.float32),
    scratch_types=[
        pltpu.VMEM((_WPW,), jnp.float32),
        pltpu.VMEM((_UC2,), jnp.int32),
        pltpu.VMEM((_UC2,), jnp.float32),
        pltpu.VMEM((_UC2,), jnp.float32),
        pltpu.VMEM((_UC2,), jnp.float32),
        pltpu.VMEM((_UC2,), jnp.int32),
        pltpu.VMEM((_UC2,), jnp.float32),
        pltpu.VMEM((_UC2,), jnp.float32),
        pltpu.VMEM((_UC2,), jnp.float32),
        pltpu.SemaphoreType.DMA,
        pltpu.SemaphoreType.DMA,
    ],
    compiler_params=_sc_params,
)
def _sc_coord_scatter(coords_in, kin_hbm, kx_hbm, ky_hbm, kz_hbm, coords_out,
                      win, kinv0, kxv0, kyv0, kzv0, kinv1, kxv1, kyv1, kzv1,
                      s0, s1):
    wid = lax.axis_index("s") * 2 + lax.axis_index("c")
    base_row = wid * _RPW
    wbase = wid * _WPW
    pltpu.sync_copy(coords_in.at[pl.ds(wbase, _WPW)], win)
    sems = (s0, s1)
    srcs = (kin_hbm, kx_hbm, ky_hbm, kz_hbm)
    slot_bufs = ((kinv0, kxv0, kyv0, kzv0), (kinv1, kxv1, kyv1, kzv1))

    def _bufs(slot):
        return slot_bufs[slot]

    def _start(c, slot):
        coff = pl.multiple_of(c * _UC2, _UC2)
        for src, dst in zip(srcs, _bufs(slot)):
            pltpu.async_copy(src.at[pl.ds(coff, _UC2)], dst, sems[slot])

    def _wait(c, slot):
        coff = pl.multiple_of(c * _UC2, _UC2)
        for src, dst in zip(srcs, _bufs(slot)):
            pltpu.make_async_copy(src.at[pl.ds(coff, _UC2)], dst,
                                  sems[slot]).wait()

    def _proc(slot):
        kinr, kxr, kyr, kzr = _bufs(slot)

        def vreg_body(u, carry):
            off0 = pl.multiple_of(u * 128, 128)
            for s in range(8):
                off = off0 + 16 * s
                kid = kinr[pl.ds(off, 16)]
                lr = kid - base_row
                um = plsc.bitcast(lr, jnp.uint32) < jnp.uint32(_RPW)
                b3 = lr * 3
                plsc.store_scatter(win, [b3], kxr[pl.ds(off, 16)], mask=um)
                plsc.store_scatter(win, [b3 + 1], kyr[pl.ds(off, 16)], mask=um)
                plsc.store_scatter(win, [b3 + 2], kzr[pl.ds(off, 16)], mask=um)
            return carry

        lax.fori_loop(0, _UC2 // 128, vreg_body, 0)

    npair = _NCHUNK2 // 2
    _start(0, 0)

    def pair_body(i, carry):
        c0 = i * 2
        _wait(c0, 0)
        _start(c0 + 1, 1)
        _proc(0)
        _wait(c0 + 1, 1)

        @pl.when(i < npair - 1)
        def _():
            _start(c0 + 2, 0)

        _proc(1)
        return carry

    lax.fori_loop(0, npair, pair_body, 0)
    pltpu.sync_copy(win, coords_out.at[pl.ds(wbase, _WPW)])


def _fkin_body(d_ref, x_ref, y_ref, z_ref):
    d = d_ref[...]
    x_ref[...] = d[:, 0] * jnp.cos(d[:, 3]) + jnp.sin(d[:, 6])
    y_ref[...] = d[:, 1] * jnp.cos(d[:, 4]) + jnp.sin(d[:, 7])
    z_ref[...] = d[:, 2] * jnp.cos(d[:, 5]) + jnp.sin(d[:, 8])


def kernel(masked_dofs, full_dofs, pose_coords, w, mask_idx, kin_id):
    dofs_flat = _sc_dof_scatter(full_dofs.reshape(-1), mask_idx, masked_dofs)
    dofs = dofs_flat.reshape(-1, 9)
    knodes = dofs.shape[0]
    blk = 8192
    kcx, kcy, kcz = pl.pallas_call(
        _fkin_body,
        grid=((knodes + blk - 1) // blk,),
        in_specs=[pl.BlockSpec((blk, 9), lambda i: (i, 0))],
        out_specs=[pl.BlockSpec((blk,), lambda i: (i,))] * 3,
        out_shape=[jax.ShapeDtypeStruct((knodes,), jnp.float32)] * 3,
    )(dofs)
    coords_flat = _sc_coord_scatter(
        pose_coords.reshape(-1), kin_id, kcx, kcy, kcz)
    coords = coords_flat.reshape(pose_coords.shape)
    coords = lax.optimization_barrier(coords)
    return jnp.einsum("pad,d->p", coords * coords, w)
